# baseline (speedup 1.0000x reference)
import functools

import jax
import jax.numpy as jnp
from jax import lax
from jax.experimental import pallas as pl
from jax.experimental.pallas import tpu as pltpu
from jax.experimental.pallas import tpu_sc as plsc

_N = 10000
_E = 320000
_R = 32
_NB = 8
_C = 128

_NC = 2
_NS = 16
_NW = _NC * _NS
_EPW = _E // _NW
_CHUNK = 80
_GBUF = 2
_IBUF = 3
_MBLK = 2000
_NMB = _EPW // _MBLK
_NCHUNK = _MBLK // _CHUNK
_TROWS = 624
_TAIL = _N - _NS * _TROWS

_BLK = 1000


def _xw_body(att_ref, basis_ref, root_ref, bias_ref, x_ref, o_ref, w_bank):
    nb = pl.program_id(0)
    r = pl.program_id(1)

    @pl.when(nb == 0)
    def _compute_w():
        @pl.when(r < _R)
        def _():
            rc = jnp.minimum(r, _R - 1)
            acc = att_ref[rc, 0] * basis_ref[0]
            for b in range(1, _NB):
                acc = acc + att_ref[rc, b] * basis_ref[b]
            w_bank[r] = acc

        @pl.when(r == _R)
        def _():
            w_bank[_R] = root_ref[...]

    y = jnp.dot(x_ref[...], w_bank[r],
                preferred_element_type=jnp.float32)
    is_root = jnp.where(r == _R, 1.0, 0.0).astype(jnp.float32)
    o_ref[0] = y + is_root * bias_ref[...]


def _xw_call(att, basis, root, bias_row, x):
    return pl.pallas_call(
        _xw_body,
        grid=(_N // _BLK, _R + 1),
        in_specs=[
            pl.BlockSpec((_R, _NB), lambda nb, r: (0, 0)),
            pl.BlockSpec((_NB, _C, _C), lambda nb, r: (0, 0, 0)),
            pl.BlockSpec((_C, _C), lambda nb, r: (0, 0)),
            pl.BlockSpec((1, _C), lambda nb, r: (0, 0)),
            pl.BlockSpec((_BLK, _C), lambda nb, r: (nb, 0)),
        ],
        out_specs=pl.BlockSpec((1, _BLK, _C), lambda nb, r: (r, nb, 0)),
        out_shape=jax.ShapeDtypeStruct((_R + 1, _N, _C), jnp.float32),
        scratch_shapes=[pltpu.VMEM((_R + 1, _C, _C), jnp.float32)],
    )(att, basis, root, bias_row, x)


def _edge_body(xw_hbm, src_hbm, type_hbm, dst_hbm, norm_hbm, zeros_hbm,
               out_hbm, src_v, type_v, dst_v, norm_v, gidx, didx, rows,
               srows, acc_sh, gsem, ssem):
    cid = lax.axis_index("c")
    sid = lax.axis_index("s")
    wid = sid * _NC + cid
    base = wid * _EPW
    trow = sid * _TROWS

    pltpu.sync_copy(zeros_hbm, acc_sh.at[pl.ds(trow, _TROWS)])

    @pl.when(sid == _NS - 1)
    def _zero_tail():
        pltpu.sync_copy(zeros_hbm.at[pl.ds(0, _TAIL)],
                        acc_sh.at[pl.ds(_NS * _TROWS, _TAIL)])

    plsc.subcore_barrier()

    iota16 = lax.iota(jnp.int32, 16)
    cols = [iota16 + s * 16 for s in range(_C // 16)]

    def _build_idx(k, slot):
        off = k * _CHUNK
        srow = jnp.full((16,), slot, jnp.int32)
        for g in range(_CHUNK // 16):
            s16 = src_v[pl.ds(off + g * 16, 16)]
            t16 = type_v[pl.ds(off + g * 16, 16)]
            d16 = dst_v[pl.ds(off + g * 16, 16)]
            plsc.store_scatter(gidx, [srow, cols[g]], t16 * _N + s16)
            plsc.store_scatter(didx, [srow, cols[g]], d16)

    def _start_gather(islot, gslot):
        return pltpu.async_copy(xw_hbm.at[gidx.at[islot]], rows.at[gslot],
                                gsem)

    def mblock_body(m, carry):
        mbase = base + m * _MBLK
        pltpu.sync_copy(src_hbm.at[pl.ds(mbase, _MBLK)], src_v)
        pltpu.sync_copy(type_hbm.at[pl.ds(mbase, _MBLK)], type_v)
        pltpu.sync_copy(dst_hbm.at[pl.ds(mbase, _MBLK)], dst_v)
        pltpu.sync_copy(norm_hbm.at[pl.ds(mbase, _MBLK)], norm_v)

        _build_idx(0, 0)
        _start_gather(0, 0)

        def chunk_body(k, c1):
            gslot = lax.rem(k, 2)
            islot = lax.rem(k, 3)

            pltpu.make_async_copy(
                xw_hbm.at[gidx.at[islot]], rows.at[gslot], gsem).wait()

            @pl.when(k + 1 < _NCHUNK)
            def _prefetch():
                _build_idx(k + 1, lax.rem(k + 1, 3))
                _start_gather(lax.rem(k + 1, 3), lax.rem(k + 1, 2))

            off = k * _CHUNK
            srow = jnp.full((16,), gslot, jnp.int32)

            @plsc.parallel_loop(0, _CHUNK, 1, unroll=4)
            def _scale(e):
                gbase = (e >> 4) << 4
                norm16 = norm_v[pl.ds(off + gbase, 16)]
                nsp = lax.gather(
                    norm16,
                    jnp.full((16, 1), e - gbase, jnp.int32),
                    lax.GatherDimensionNumbers(
                        offset_dims=(),
                        collapsed_slice_dims=(0,),
                        start_index_map=(0,)),
                    (1,),
                    mode=lax.GatherScatterMode.PROMISE_IN_BOUNDS)
                erow = jnp.full((16,), e, jnp.int32)
                for s in range(_C // 16):
                    v = plsc.load_gather(rows, [srow, erow, cols[s]])
                    plsc.store_scatter(srows, [srow, erow, cols[s]],
                                       v * nsp)

            pltpu.sync_copy(srows.at[gslot], acc_sh.at[didx.at[islot]],
                            add=True)

            return c1

        lax.fori_loop(0, _NCHUNK, chunk_body, 0)
        return carry

    lax.fori_loop(0, _NMB, mblock_body, 0)

    plsc.subcore_barrier()

    pltpu.sync_copy(acc_sh.at[pl.ds(trow, _TROWS)],
                    out_hbm.at[cid, pl.ds(trow, _TROWS)])

    @pl.when(sid == _NS - 1)
    def _flush_tail():
        pltpu.sync_copy(acc_sh.at[pl.ds(_NS * _TROWS, _TAIL)],
                        out_hbm.at[cid, pl.ds(_NS * _TROWS, _TAIL)])


@functools.cache
def _edge_call():
    return pl.kernel(
        _edge_body,
        out_type=jax.ShapeDtypeStruct((_NC, _N, _C), jnp.float32),
        mesh=plsc.VectorSubcoreMesh(core_axis_name="c", subcore_axis_name="s",
                                    num_cores=_NC, num_subcores=_NS),
        compiler_params=pltpu.CompilerParams(needs_layout_passes=False),
        scratch_types=[
            pltpu.VMEM((_MBLK,), jnp.int32),
            pltpu.VMEM((_MBLK,), jnp.int32),
            pltpu.VMEM((_MBLK,), jnp.int32),
            pltpu.VMEM((_MBLK,), jnp.float32),
            pltpu.VMEM((_IBUF, _CHUNK), jnp.int32),
            pltpu.VMEM((_IBUF, _CHUNK), jnp.int32),
            pltpu.VMEM((_GBUF, _CHUNK, _C), jnp.float32),
            pltpu.VMEM((_GBUF, _CHUNK, _C), jnp.float32),
            pltpu.VMEM_SHARED((_N, _C), jnp.float32),
            pltpu.SemaphoreType.DMA,
            pltpu.SemaphoreType.DMA,
        ],
    )


def _combine_body(p0_ref, p1_ref, b_ref, o_ref):
    o_ref[...] = p0_ref[...] + p1_ref[...] + b_ref[...]


def _combine_call(p0, p1, base):
    return pl.pallas_call(
        _combine_body,
        grid=(_N // _BLK,),
        in_specs=[
            pl.BlockSpec((_BLK, _C), lambda nb: (nb, 0)),
            pl.BlockSpec((_BLK, _C), lambda nb: (nb, 0)),
            pl.BlockSpec((_BLK, _C), lambda nb: (nb, 0)),
        ],
        out_specs=pl.BlockSpec((_BLK, _C), lambda nb: (nb, 0)),
        out_shape=jax.ShapeDtypeStruct((_N, _C), jnp.float32),
    )(p0, p1, base)


@jax.jit
def kernel(x, edge_index, edge_type, edge_norm, basis, att, root, bias):
    xw_all = _xw_call(att, basis, root, bias.reshape(1, _C), x)
    xw_flat = xw_all.reshape((_R + 1) * _N, _C)
    zeros = jnp.zeros((_TROWS, _C), jnp.float32)
    partials = _edge_call()(xw_flat, edge_index[0], edge_type,
                            edge_index[1], edge_norm, zeros)
    return _combine_call(partials[0], partials[1], xw_all[_R])


# speedup vs baseline: 1.0073x; 1.0073x over previous
import functools

import jax
import jax.numpy as jnp
from jax import lax
from jax.experimental import pallas as pl
from jax.experimental.pallas import tpu as pltpu
from jax.experimental.pallas import tpu_sc as plsc

_N = 10000
_E = 320000
_R = 32
_NB = 8
_C = 128

_NC = 2
_NS = 16
_NW = _NC * _NS
_EPW = _E // _NW
_CHUNK = 80
_GBUF = 2
_IBUF = 3
_MBLK = 2000
_NMB = _EPW // _MBLK
_NCHUNK = _MBLK // _CHUNK
_TROWS = 624
_TAIL = _N - _NS * _TROWS

_BLK = 1000


def _xw_body(att_ref, basis_ref, root_ref, bias_ref, x_ref, o_ref, w_bank):
    nb = pl.program_id(0)
    r = pl.program_id(1)

    @pl.when(nb == 0)
    def _compute_w():
        @pl.when(r < _R)
        def _():
            rc = jnp.minimum(r, _R - 1)
            acc = att_ref[rc, 0] * basis_ref[0]
            for b in range(1, _NB):
                acc = acc + att_ref[rc, b] * basis_ref[b]
            w_bank[r] = acc.astype(jnp.bfloat16)

        @pl.when(r == _R)
        def _():
            w_bank[_R] = root_ref[...].astype(jnp.bfloat16)

    y = jnp.dot(x_ref[...].astype(jnp.bfloat16), w_bank[r],
                preferred_element_type=jnp.float32)
    is_root = jnp.where(r == _R, 1.0, 0.0).astype(jnp.float32)
    o_ref[0] = y + is_root * bias_ref[...]


def _xw_call(att, basis, root, bias_row, x):
    return pl.pallas_call(
        _xw_body,
        grid=(_N // _BLK, _R + 1),
        in_specs=[
            pl.BlockSpec((_R, _NB), lambda nb, r: (0, 0)),
            pl.BlockSpec((_NB, _C, _C), lambda nb, r: (0, 0, 0)),
            pl.BlockSpec((_C, _C), lambda nb, r: (0, 0)),
            pl.BlockSpec((1, _C), lambda nb, r: (0, 0)),
            pl.BlockSpec((_BLK, _C), lambda nb, r: (nb, 0)),
        ],
        out_specs=pl.BlockSpec((1, _BLK, _C), lambda nb, r: (r, nb, 0)),
        out_shape=jax.ShapeDtypeStruct((_R + 1, _N, _C), jnp.float32),
        scratch_shapes=[pltpu.VMEM((_R + 1, _C, _C), jnp.bfloat16)],
    )(att, basis, root, bias_row, x)


def _edge_body(xw_hbm, src_hbm, type_hbm, dst_hbm, norm_hbm, zeros_hbm,
               out_hbm, src_v, type_v, dst_v, norm_v, gidx, didx, rows,
               srows, acc_sh, gsem, ssem):
    cid = lax.axis_index("c")
    sid = lax.axis_index("s")
    wid = sid * _NC + cid
    base = wid * _EPW
    trow = sid * _TROWS

    pltpu.sync_copy(zeros_hbm, acc_sh.at[pl.ds(trow, _TROWS)])

    @pl.when(sid == _NS - 1)
    def _zero_tail():
        pltpu.sync_copy(zeros_hbm.at[pl.ds(0, _TAIL)],
                        acc_sh.at[pl.ds(_NS * _TROWS, _TAIL)])

    plsc.subcore_barrier()

    iota16 = lax.iota(jnp.int32, 16)
    cols = [iota16 + s * 16 for s in range(_C // 16)]

    def _build_idx(k, slot):
        off = k * _CHUNK
        srow = jnp.full((16,), slot, jnp.int32)
        for g in range(_CHUNK // 16):
            s16 = src_v[pl.ds(off + g * 16, 16)]
            t16 = type_v[pl.ds(off + g * 16, 16)]
            d16 = dst_v[pl.ds(off + g * 16, 16)]
            plsc.store_scatter(gidx, [srow, cols[g]], t16 * _N + s16)
            plsc.store_scatter(didx, [srow, cols[g]], d16)

    def _start_gather(islot, gslot):
        return pltpu.async_copy(xw_hbm.at[gidx.at[islot]], rows.at[gslot],
                                gsem)

    def mblock_body(m, carry):
        mbase = base + m * _MBLK
        pltpu.sync_copy(src_hbm.at[pl.ds(mbase, _MBLK)], src_v)
        pltpu.sync_copy(type_hbm.at[pl.ds(mbase, _MBLK)], type_v)
        pltpu.sync_copy(dst_hbm.at[pl.ds(mbase, _MBLK)], dst_v)
        pltpu.sync_copy(norm_hbm.at[pl.ds(mbase, _MBLK)], norm_v)

        _build_idx(0, 0)
        _start_gather(0, 0)

        def chunk_body(k, c1):
            gslot = lax.rem(k, 2)
            islot = lax.rem(k, 3)

            pltpu.make_async_copy(
                xw_hbm.at[gidx.at[islot]], rows.at[gslot], gsem).wait()

            @pl.when(k + 1 < _NCHUNK)
            def _prefetch():
                _build_idx(k + 1, lax.rem(k + 1, 3))
                _start_gather(lax.rem(k + 1, 3), lax.rem(k + 1, 2))

            off = k * _CHUNK
            srow = jnp.full((16,), gslot, jnp.int32)

            @plsc.parallel_loop(0, _CHUNK, 1, unroll=4)
            def _scale(e):
                gbase = (e >> 4) << 4
                norm16 = norm_v[pl.ds(off + gbase, 16)]
                nsp = lax.gather(
                    norm16,
                    jnp.full((16, 1), e - gbase, jnp.int32),
                    lax.GatherDimensionNumbers(
                        offset_dims=(),
                        collapsed_slice_dims=(0,),
                        start_index_map=(0,)),
                    (1,),
                    mode=lax.GatherScatterMode.PROMISE_IN_BOUNDS)
                erow = jnp.full((16,), e, jnp.int32)
                for s in range(_C // 16):
                    v = plsc.load_gather(rows, [srow, erow, cols[s]])
                    plsc.store_scatter(srows, [srow, erow, cols[s]],
                                       v * nsp)

            pltpu.sync_copy(srows.at[gslot], acc_sh.at[didx.at[islot]],
                            add=True)

            return c1

        lax.fori_loop(0, _NCHUNK, chunk_body, 0)
        return carry

    lax.fori_loop(0, _NMB, mblock_body, 0)

    plsc.subcore_barrier()

    pltpu.sync_copy(acc_sh.at[pl.ds(trow, _TROWS)],
                    out_hbm.at[cid, pl.ds(trow, _TROWS)])

    @pl.when(sid == _NS - 1)
    def _flush_tail():
        pltpu.sync_copy(acc_sh.at[pl.ds(_NS * _TROWS, _TAIL)],
                        out_hbm.at[cid, pl.ds(_NS * _TROWS, _TAIL)])


@functools.cache
def _edge_call():
    return pl.kernel(
        _edge_body,
        out_type=jax.ShapeDtypeStruct((_NC, _N, _C), jnp.float32),
        mesh=plsc.VectorSubcoreMesh(core_axis_name="c", subcore_axis_name="s",
                                    num_cores=_NC, num_subcores=_NS),
        compiler_params=pltpu.CompilerParams(needs_layout_passes=False),
        scratch_types=[
            pltpu.VMEM((_MBLK,), jnp.int32),
            pltpu.VMEM((_MBLK,), jnp.int32),
            pltpu.VMEM((_MBLK,), jnp.int32),
            pltpu.VMEM((_MBLK,), jnp.float32),
            pltpu.VMEM((_IBUF, _CHUNK), jnp.int32),
            pltpu.VMEM((_IBUF, _CHUNK), jnp.int32),
            pltpu.VMEM((_GBUF, _CHUNK, _C), jnp.float32),
            pltpu.VMEM((_GBUF, _CHUNK, _C), jnp.float32),
            pltpu.VMEM_SHARED((_N, _C), jnp.float32),
            pltpu.SemaphoreType.DMA,
            pltpu.SemaphoreType.DMA,
        ],
    )


def _combine_body(p0_ref, p1_ref, b_ref, o_ref):
    o_ref[...] = p0_ref[...] + p1_ref[...] + b_ref[...]


def _combine_call(p0, p1, base):
    return pl.pallas_call(
        _combine_body,
        grid=(_N // _BLK,),
        in_specs=[
            pl.BlockSpec((_BLK, _C), lambda nb: (nb, 0)),
            pl.BlockSpec((_BLK, _C), lambda nb: (nb, 0)),
            pl.BlockSpec((_BLK, _C), lambda nb: (nb, 0)),
        ],
        out_specs=pl.BlockSpec((_BLK, _C), lambda nb: (nb, 0)),
        out_shape=jax.ShapeDtypeStruct((_N, _C), jnp.float32),
    )(p0, p1, base)


@jax.jit
def kernel(x, edge_index, edge_type, edge_norm, basis, att, root, bias):
    xw_all = _xw_call(att, basis, root, bias.reshape(1, _C), x)
    xw_flat = xw_all.reshape((_R + 1) * _N, _C)
    zeros = jnp.zeros((_TROWS, _C), jnp.float32)
    partials = _edge_call()(xw_flat, edge_index[0], edge_type,
                            edge_index[1], edge_norm, zeros)
    return _combine_call(partials[0], partials[1], xw_all[_R])


# speedup vs baseline: 1.2067x; 1.1979x over previous
import functools

import jax
import jax.numpy as jnp
from jax import lax
from jax.experimental import pallas as pl
from jax.experimental.pallas import tpu as pltpu
from jax.experimental.pallas import tpu_sc as plsc

_N = 10000
_E = 320000
_R = 32
_NB = 8
_C = 128

_NC = 2
_NS = 16
_NW = _NC * _NS
_EPW = _E // _NW
_CHUNK = 80
_GBUF = 2
_IBUF = 3
_MBLK = 2000
_NMB = _EPW // _MBLK
_NCHUNK = _MBLK // _CHUNK
_TROWS = 624
_TAIL = _N - _NS * _TROWS

_BLK = 2000


def _xw_body(att_ref, basis_ref, root_ref, bias_ref, x_ref, o_ref, w_bank):
    nb = pl.program_id(0)
    r = pl.program_id(1)

    @pl.when(nb == 0)
    def _compute_w():
        @pl.when(r < _R)
        def _():
            rc = jnp.minimum(r, _R - 1)
            acc = att_ref[rc, 0] * basis_ref[0]
            for b in range(1, _NB):
                acc = acc + att_ref[rc, b] * basis_ref[b]
            w_bank[r] = acc.astype(jnp.bfloat16)

        @pl.when(r == _R)
        def _():
            w_bank[_R] = root_ref[...].astype(jnp.bfloat16)

    y = jnp.dot(x_ref[...], w_bank[r], preferred_element_type=jnp.float32)
    o_ref[0] = y

    @pl.when(r == _R)
    def _add_bias():
        o_ref[0] = y + bias_ref[...]


def _xw_call(att, basis, root, bias_row, x):
    return pl.pallas_call(
        _xw_body,
        grid=(_N // _BLK, _R + 1),
        in_specs=[
            pl.BlockSpec((_R, _NB), lambda nb, r: (0, 0)),
            pl.BlockSpec((_NB, _C, _C), lambda nb, r: (0, 0, 0)),
            pl.BlockSpec((_C, _C), lambda nb, r: (0, 0)),
            pl.BlockSpec((1, _C), lambda nb, r: (0, 0)),
            pl.BlockSpec((_BLK, _C), lambda nb, r: (nb, 0)),
        ],
        out_specs=pl.BlockSpec((1, _BLK, _C), lambda nb, r: (r, nb, 0)),
        out_shape=jax.ShapeDtypeStruct((_R + 1, _N, _C), jnp.float32),
        scratch_shapes=[pltpu.VMEM((_R + 1, _C, _C), jnp.bfloat16)],
    )(att, basis, root, bias_row, x)


def _edge_body(xw_hbm, src_hbm, type_hbm, dst_hbm, norm_hbm, zeros_hbm,
               out_hbm, src_v, type_v, dst_v, norm_v, gidx, didx, rows,
               srows, acc_sh, gsem, ssem):
    cid = lax.axis_index("c")
    sid = lax.axis_index("s")
    wid = sid * _NC + cid
    base = wid * _EPW
    trow = sid * _TROWS

    pltpu.sync_copy(zeros_hbm, acc_sh.at[pl.ds(trow, _TROWS)])

    @pl.when(sid == _NS - 1)
    def _zero_tail():
        pltpu.sync_copy(zeros_hbm.at[pl.ds(0, _TAIL)],
                        acc_sh.at[pl.ds(_NS * _TROWS, _TAIL)])

    plsc.subcore_barrier()

    iota16 = lax.iota(jnp.int32, 16)
    cols = [iota16 + s * 16 for s in range(_C // 16)]

    def _build_idx(k, slot):
        off = k * _CHUNK
        srow = jnp.full((16,), slot, jnp.int32)
        for g in range(_CHUNK // 16):
            s16 = src_v[pl.ds(off + g * 16, 16)]
            t16 = type_v[pl.ds(off + g * 16, 16)]
            d16 = dst_v[pl.ds(off + g * 16, 16)]
            plsc.store_scatter(gidx, [srow, cols[g]], t16 * _N + s16)
            plsc.store_scatter(didx, [srow, cols[g]], d16)

    def _start_gather(islot, gslot):
        return pltpu.async_copy(xw_hbm.at[gidx.at[islot]], rows.at[gslot],
                                gsem)

    def mblock_body(m, carry):
        mbase = base + m * _MBLK
        pltpu.sync_copy(src_hbm.at[pl.ds(mbase, _MBLK)], src_v)
        pltpu.sync_copy(type_hbm.at[pl.ds(mbase, _MBLK)], type_v)
        pltpu.sync_copy(dst_hbm.at[pl.ds(mbase, _MBLK)], dst_v)
        pltpu.sync_copy(norm_hbm.at[pl.ds(mbase, _MBLK)], norm_v)

        _build_idx(0, 0)
        _start_gather(0, 0)

        def chunk_body(k, c1):
            gslot = lax.rem(k, 2)
            islot = lax.rem(k, 3)

            pltpu.make_async_copy(
                xw_hbm.at[gidx.at[islot]], rows.at[gslot], gsem).wait()

            @pl.when(k + 1 < _NCHUNK)
            def _prefetch():
                _build_idx(k + 1, lax.rem(k + 1, 3))
                _start_gather(lax.rem(k + 1, 3), lax.rem(k + 1, 2))

            off = k * _CHUNK
            srow = jnp.full((16,), gslot, jnp.int32)

            @plsc.parallel_loop(0, _CHUNK, 1, unroll=4)
            def _scale(e):
                gbase = (e >> 4) << 4
                norm16 = norm_v[pl.ds(off + gbase, 16)]
                nsp = lax.gather(
                    norm16,
                    jnp.full((16, 1), e - gbase, jnp.int32),
                    lax.GatherDimensionNumbers(
                        offset_dims=(),
                        collapsed_slice_dims=(0,),
                        start_index_map=(0,)),
                    (1,),
                    mode=lax.GatherScatterMode.PROMISE_IN_BOUNDS)
                erow = jnp.full((16,), e, jnp.int32)
                for s in range(_C // 16):
                    v = plsc.load_gather(rows, [srow, erow, cols[s]])
                    plsc.store_scatter(srows, [srow, erow, cols[s]],
                                       v * nsp)

            pltpu.sync_copy(srows.at[gslot], acc_sh.at[didx.at[islot]],
                            add=True)

            return c1

        lax.fori_loop(0, _NCHUNK, chunk_body, 0)
        return carry

    lax.fori_loop(0, _NMB, mblock_body, 0)

    plsc.subcore_barrier()

    pltpu.sync_copy(acc_sh.at[pl.ds(trow, _TROWS)],
                    out_hbm.at[cid, pl.ds(trow, _TROWS)])

    @pl.when(sid == _NS - 1)
    def _flush_tail():
        pltpu.sync_copy(acc_sh.at[pl.ds(_NS * _TROWS, _TAIL)],
                        out_hbm.at[cid, pl.ds(_NS * _TROWS, _TAIL)])


@functools.cache
def _edge_call():
    return pl.kernel(
        _edge_body,
        out_type=jax.ShapeDtypeStruct((_NC, _N, _C), jnp.float32),
        mesh=plsc.VectorSubcoreMesh(core_axis_name="c", subcore_axis_name="s",
                                    num_cores=_NC, num_subcores=_NS),
        compiler_params=pltpu.CompilerParams(needs_layout_passes=False),
        scratch_types=[
            pltpu.VMEM((_MBLK,), jnp.int32),
            pltpu.VMEM((_MBLK,), jnp.int32),
            pltpu.VMEM((_MBLK,), jnp.int32),
            pltpu.VMEM((_MBLK,), jnp.float32),
            pltpu.VMEM((_IBUF, _CHUNK), jnp.int32),
            pltpu.VMEM((_IBUF, _CHUNK), jnp.int32),
            pltpu.VMEM((_GBUF, _CHUNK, _C), jnp.float32),
            pltpu.VMEM((_GBUF, _CHUNK, _C), jnp.float32),
            pltpu.VMEM_SHARED((_N, _C), jnp.float32),
            pltpu.SemaphoreType.DMA,
            pltpu.SemaphoreType.DMA,
        ],
    )


def _combine_body(p0_ref, p1_ref, b_ref, o_ref):
    o_ref[...] = p0_ref[...] + p1_ref[...] + b_ref[...]


def _combine_call(p0, p1, base):
    return pl.pallas_call(
        _combine_body,
        grid=(_N // _BLK,),
        in_specs=[
            pl.BlockSpec((_BLK, _C), lambda nb: (nb, 0)),
            pl.BlockSpec((_BLK, _C), lambda nb: (nb, 0)),
            pl.BlockSpec((_BLK, _C), lambda nb: (nb, 0)),
        ],
        out_specs=pl.BlockSpec((_BLK, _C), lambda nb: (nb, 0)),
        out_shape=jax.ShapeDtypeStruct((_N, _C), jnp.float32),
    )(p0, p1, base)


@jax.jit
def kernel(x, edge_index, edge_type, edge_norm, basis, att, root, bias):
    xw_all = _xw_call(att, basis, root, bias.reshape(1, _C),
                      x.astype(jnp.bfloat16))
    xw_flat = xw_all.reshape((_R + 1) * _N, _C)
    zeros = jnp.zeros((_TROWS, _C), jnp.float32)
    partials = _edge_call()(xw_flat, edge_index[0], edge_type,
                            edge_index[1], edge_norm, zeros)
    return _combine_call(partials[0], partials[1], xw_all[_R])
